# Initial kernel scaffold; baseline (speedup 1.0000x reference)
import jax
import jax.numpy as jnp
from jax.experimental import pallas as pl


def kernel(x_indices, ei, emb_table, W, b):
    raise NotImplementedError("write your pallas kernel here")


# speedup vs baseline: 11.9992x; 11.9992x over previous
import functools

import jax
import jax.numpy as jnp
from jax import lax
from jax.experimental import pallas as pl
from jax.experimental.pallas import tpu as pltpu
from jax.experimental.pallas import tpu_sc as plsc

N = 10000
E = 320000
D = 128
OUT_C = 128

NC = 2
NS = 16
L = 16
NW = NC * NS

CHUNK = 128
CPT = -(-E // (NW * CHUNK))
EPT = CPT * CHUNK
E_PAD = NW * EPT

RPT = 640
N_PAD = NS * RPT

XPT = 320
XC = 3
XW = (CHUNK, CHUNK, XPT - 2 * CHUNK)

_mesh = plsc.VectorSubcoreMesh(core_axis_name="c", subcore_axis_name="s")
_f32 = jnp.float32


def _deg_gather_body(eint_hbm, xidx_hbm, emb_hbm, degp_hbm, xg_hbm,
                     idx_v, xidx_v, ones_v, xrows_v, deg_sh, sem):
    c = lax.axis_index("c")
    s = lax.axis_index("s")
    w = c * NS + s

    def _fill(val):
        def body(i, carry):
            ones_v[i, :] = jnp.full((L,), val, _f32)
            return carry
        lax.fori_loop(0, CHUNK, body, 0)

    _fill(0.0)
    for k in range(RPT // CHUNK):
        pltpu.sync_copy(ones_v, deg_sh.at[pl.ds(s * RPT + k * CHUNK, CHUNK)])
    _fill(1.0)

    pltpu.sync_copy(xidx_hbm.at[w], xidx_v)
    for k in range(XC):
        pltpu.async_copy(emb_hbm.at[xidx_v.at[k]], xrows_v, sem).wait()
        pltpu.sync_copy(xrows_v.at[pl.ds(0, XW[k])],
                        xg_hbm.at[pl.ds(w * XPT + k * CHUNK, XW[k])])

    plsc.subcore_barrier()

    pltpu.sync_copy(eint_hbm.at[w, 0], idx_v.at[0])

    def _deg_loop(j, carry):
        cur = lax.rem(j, 2)

        @pl.when(j + 1 < CPT)
        def _fetch_next():
            pltpu.async_copy(eint_hbm.at[w, j + 1], idx_v.at[1 - cur], sem)

        pltpu.sync_copy(ones_v, deg_sh.at[idx_v.at[cur, 1]], add=True)

        @pl.when(j + 1 < CPT)
        def _wait_next():
            pltpu.make_async_copy(eint_hbm.at[w, j + 1],
                                  idx_v.at[1 - cur], sem).wait()
        return carry
    lax.fori_loop(0, CPT, _deg_loop, 0)

    plsc.subcore_barrier()
    pltpu.sync_copy(deg_sh.at[pl.ds(s * RPT, RPT)],
                    degp_hbm.at[c, pl.ds(s * RPT, RPT)])


_deg_gather = functools.partial(
    pl.kernel,
    out_type=[
        jax.ShapeDtypeStruct((NC, N_PAD, L), _f32),
        jax.ShapeDtypeStruct((N_PAD, D), _f32),
    ],
    mesh=_mesh,
    scratch_types=[
        pltpu.VMEM((2, 2, CHUNK), jnp.int32),
        pltpu.VMEM((XC, CHUNK), jnp.int32),
        pltpu.VMEM((CHUNK, L), _f32),
        pltpu.VMEM((CHUNK, D), _f32),
        pltpu.VMEM_SHARED((N_PAD, L), _f32),
        pltpu.SemaphoreType.DMA,
    ],
)(_deg_gather_body)


def _hop_body(eint_hbm, y_hbm, out_hbm, idx_v, rows_v, acc_sh, sem_i, sem_g):
    c = lax.axis_index("c")
    s = lax.axis_index("s")
    w = c * NS + s

    def _z(i, carry):
        r = i // (D // L)
        l = i % (D // L)
        rows_v[0, r, pl.ds(l * L, L)] = jnp.zeros((L,), _f32)
        return carry
    lax.fori_loop(0, CHUNK * (D // L), _z, 0)

    for k in range(RPT // CHUNK):
        pltpu.sync_copy(rows_v.at[0],
                        acc_sh.at[pl.ds(s * RPT + k * CHUNK, CHUNK)])

    plsc.subcore_barrier()

    pltpu.sync_copy(eint_hbm.at[w, 0], idx_v.at[0])
    pltpu.async_copy(y_hbm.at[idx_v.at[0, 0]], rows_v.at[0], sem_g)

    def _loop(j, carry):
        cur = lax.rem(j, 2)
        nxt = 1 - cur

        @pl.when(j + 1 < CPT)
        def _fetch_idx():
            pltpu.async_copy(eint_hbm.at[w, j + 1], idx_v.at[nxt], sem_i)
            pltpu.make_async_copy(eint_hbm.at[w, j + 1],
                                  idx_v.at[nxt], sem_i).wait()

        pltpu.make_async_copy(y_hbm.at[idx_v.at[cur, 0]],
                              rows_v.at[cur], sem_g).wait()

        @pl.when(j + 1 < CPT)
        def _gather_next():
            pltpu.async_copy(y_hbm.at[idx_v.at[nxt, 0]], rows_v.at[nxt], sem_g)

        pltpu.sync_copy(rows_v.at[cur], acc_sh.at[idx_v.at[cur, 1]], add=True)
        return carry
    lax.fori_loop(0, CPT, _loop, 0)

    plsc.subcore_barrier()
    pltpu.sync_copy(acc_sh.at[pl.ds(s * RPT, RPT)],
                    out_hbm.at[c, pl.ds(s * RPT, RPT)])


_hop = functools.partial(
    pl.kernel,
    out_type=jax.ShapeDtypeStruct((NC, N_PAD, D), _f32),
    mesh=_mesh,
    scratch_types=[
        pltpu.VMEM((2, 2, CHUNK), jnp.int32),
        pltpu.VMEM((2, CHUNK, D), _f32),
        pltpu.VMEM_SHARED((N_PAD, D), _f32),
        pltpu.SemaphoreType.DMA,
        pltpu.SemaphoreType.DMA,
    ],
)(_hop_body)


def _scale1_body(xg_ref, degp_ref, y_ref, dinv_ref):
    deg = degp_ref[0][:, 0:1] + degp_ref[1][:, 0:1] + 1.0
    dinv = lax.rsqrt(deg)
    dinv_ref[...] = dinv
    y_ref[...] = xg_ref[...] * dinv


_scale1 = pl.pallas_call(
    _scale1_body,
    out_shape=[
        jax.ShapeDtypeStruct((N_PAD, D), _f32),
        jax.ShapeDtypeStruct((N_PAD, 1), _f32),
    ],
)


def _scale2_body(p_ref, y_ref, dinv_ref, z_ref):
    d = dinv_ref[...]
    z_ref[...] = (p_ref[0] + p_ref[1] + y_ref[...]) * (d * d)


_scale2 = pl.pallas_call(
    _scale2_body,
    out_shape=jax.ShapeDtypeStruct((N_PAD, D), _f32),
)


def _final_body(q_ref, z_ref, dinv_ref, w_ref, b_ref, o_ref):
    h = (q_ref[0] + q_ref[1] + z_ref[...]) * dinv_ref[...]
    o_ref[...] = lax.dot_general(
        h, w_ref[...], (((1,), (1,)), ((), ())),
        preferred_element_type=_f32) + b_ref[...]


_final = pl.pallas_call(
    _final_body,
    out_shape=jax.ShapeDtypeStruct((N_PAD, OUT_C), _f32),
)


@jax.jit
def kernel(x_indices, ei, emb_table, W, b):
    i32 = jnp.int32
    src_p = jnp.concatenate(
        [ei[0], jnp.zeros((E_PAD - E,), i32)]).reshape(NW, CPT, CHUNK)
    dst_p = jnp.concatenate(
        [ei[1], jnp.full((E_PAD - E,), N, i32)]).reshape(NW, CPT, CHUNK)
    eint = jnp.stack([src_p, dst_p], axis=2)
    xi = jnp.concatenate(
        [x_indices.astype(i32), jnp.zeros((N_PAD - N,), i32)]).reshape(NW, XPT)
    xi_p = jnp.zeros((NW, XC * CHUNK), i32).at[:, :XPT].set(xi).reshape(
        NW, XC, CHUNK)

    degp, xg = _deg_gather(eint, xi_p, emb_table)
    y, dinv = _scale1(xg, degp)
    p = _hop(eint, y)
    z = _scale2(p, y, dinv)
    q = _hop(eint, z)
    out = _final(q, z, dinv, W, b.reshape(1, OUT_C))
    return out[:N]
